# baseline (speedup 1.0000x reference)
import functools

import jax
import jax.numpy as jnp
from jax import lax
from jax.experimental import pallas as pl
from jax.experimental.pallas import tpu as pltpu
from jax.experimental.pallas import tpu_sc as plsc

N = 10000
E = 320000
D = 128

NC = 2
NS = 16
LANES = 128
CHUNKS = 80
E_PAD = NC * NS * CHUNKS * LANES
ROWS_PER_TILE = 632
N_ACC = NC * NS * ROWS_PER_TILE // 2
PAD_DST_ROWS = 64


K = 2


def _make_agg(with_deg: bool):
    out_type = [jax.ShapeDtypeStruct((NC, N_ACC, D), jnp.float32)]
    if with_deg:
        out_type.append(jax.ShapeDtypeStruct((NC * N_ACC,), jnp.float32))

    scratch = [
        pltpu.VMEM((CHUNKS // 2, LANES), jnp.int32),
        pltpu.VMEM((CHUNKS // 2, LANES), jnp.int32),
        pltpu.VMEM((K, LANES, D), jnp.float32),
        pltpu.VMEM((K, LANES), jnp.float32),
        pltpu.VMEM((640,), jnp.float32),
        pltpu.VMEM_SHARED((N_ACC, D), jnp.float32),
        pltpu.VMEM_SHARED((N_ACC,), jnp.float32),
        pltpu.SemaphoreType.DMA,
        pltpu.SemaphoreType.DMA,
        pltpu.SemaphoreType.DMA,
        pltpu.SemaphoreType.DMA,
        pltpu.SemaphoreType.DMA,
    ]

    def body(src_hbm, dst_hbm, feat_hbm, *rest):
        if with_deg:
            out_hbm, deg_hbm = rest[0], rest[1]
            scratches = rest[2:]
        else:
            out_hbm = rest[0]
            scratches = rest[1:]
        (src_v, dst_v, rows_kv, ones_v, dzero_v, acc_sh, deg_sh,
         gsem0, gsem1, ssem0, ssem1, dsem) = scratches
        rows_v = rows_kv.at[0]

        cid = lax.axis_index("c")
        sid = lax.axis_index("s")
        tid = cid * NS + sid

        def zrow(i, _):
            for j in range(D // 16):
                rows_kv[0, i, pl.ds(j * 16, 16)] = jnp.zeros((16,),
                                                             jnp.float32)
            return 0
        lax.fori_loop(0, LANES, zrow, 0)
        for k in range(K):
            for j in range(LANES // 16):
                ones_v[k, pl.ds(j * 16, 16)] = jnp.ones((16,), jnp.float32)

        base = sid * ROWS_PER_TILE
        full, tail = divmod(ROWS_PER_TILE, LANES)
        for k in range(full):
            pltpu.sync_copy(rows_v, acc_sh.at[pl.ds(base + k * LANES, LANES)])
        if tail:
            pltpu.sync_copy(rows_v.at[pl.ds(0, tail)],
                            acc_sh.at[pl.ds(base + full * LANES, tail)])
        if with_deg:
            def zdeg(i, _):
                dzero_v[pl.ds(i * 16, 16)] = jnp.zeros((16,), jnp.float32)
                return 0
            lax.fori_loop(0, 640 // 16, zdeg, 0)
            pltpu.sync_copy(dzero_v.at[pl.ds(0, ROWS_PER_TILE)],
                            deg_sh.at[pl.ds(base, ROWS_PER_TILE)])
        plsc.subcore_barrier()

        HALF = CHUNKS // 2
        NSTEP = HALF // K

        for h in range(2):
            pltpu.sync_copy(src_hbm.at[tid, pl.ds(h * HALF, HALF)], src_v)
            pltpu.sync_copy(dst_hbm.at[tid, pl.ds(h * HALF, HALF)], dst_v)

            def step(i, _):
                j = i * K
                g0 = pltpu.async_copy(feat_hbm.at[src_v.at[j]],
                                      rows_kv.at[0], gsem0)
                g1 = pltpu.async_copy(feat_hbm.at[src_v.at[j + 1]],
                                      rows_kv.at[1], gsem1)
                if with_deg:
                    d0 = pltpu.async_copy(ones_v.at[0],
                                          deg_sh.at[dst_v.at[j]], dsem,
                                          add=True)
                    d1 = pltpu.async_copy(ones_v.at[1],
                                          deg_sh.at[dst_v.at[j + 1]], dsem,
                                          add=True)
                g0.wait()
                s0 = pltpu.async_copy(rows_kv.at[0],
                                      acc_sh.at[dst_v.at[j]], ssem0,
                                      add=True)
                g1.wait()
                s1 = pltpu.async_copy(rows_kv.at[1],
                                      acc_sh.at[dst_v.at[j + 1]], ssem1,
                                      add=True)
                if with_deg:
                    d0.wait()
                    d1.wait()
                s0.wait()
                s1.wait()
                return 0

            lax.fori_loop(0, NSTEP, step, 0)
        plsc.subcore_barrier()

        pltpu.sync_copy(acc_sh.at[pl.ds(base, ROWS_PER_TILE)],
                        out_hbm.at[cid, pl.ds(base, ROWS_PER_TILE)])
        if with_deg:
            pltpu.sync_copy(deg_sh.at[pl.ds(base, ROWS_PER_TILE)],
                            dzero_v.at[pl.ds(0, ROWS_PER_TILE)])
            pltpu.sync_copy(dzero_v.at[pl.ds(0, ROWS_PER_TILE)],
                            deg_hbm.at[pl.ds(cid * N_ACC + base,
                                             ROWS_PER_TILE)])

    mesh = plsc.VectorSubcoreMesh(core_axis_name="c", subcore_axis_name="s")
    return pl.kernel(body, out_type=out_type, mesh=mesh,
                     scratch_types=scratch)


_agg_deg = _make_agg(True)
_agg_nodeg = _make_agg(False)


def _lin_body(relu, p0, p1, d0, d1, xr, wl, wr, b, o):
    deg = jnp.clip(d0[...] + d1[...], 1.0, None)
    mean = (p0[...] + p1[...]) / deg
    y = (jnp.dot(mean, wl[...], preferred_element_type=jnp.float32)
         + jnp.dot(xr[...], wr[...], preferred_element_type=jnp.float32)
         + b[...])
    o[...] = jnp.maximum(y, 0.0) if relu else y


def _linear(p0, p1, d0, d1, x, W_l, W_r, b, relu):
    B = 2000
    grid = (N // B,)
    row = lambda i: (i, 0)
    fix = lambda i: (0, 0)
    return pl.pallas_call(
        functools.partial(_lin_body, relu),
        grid=grid,
        in_specs=[
            pl.BlockSpec((B, D), row), pl.BlockSpec((B, D), row),
            pl.BlockSpec((B, 1), row), pl.BlockSpec((B, 1), row),
            pl.BlockSpec((B, D), row),
            pl.BlockSpec((D, D), fix), pl.BlockSpec((D, D), fix),
            pl.BlockSpec((1, D), fix),
        ],
        out_specs=pl.BlockSpec((B, D), row),
        out_shape=jax.ShapeDtypeStruct((N, D), jnp.float32),
    )(p0, p1, d0, d1, x, W_l, W_r, b.reshape(1, D))


def kernel(x, edge_index, W1_l, W1_r, b1, W2_l, W2_r, b2):
    src = edge_index[0]
    dst = edge_index[1]
    pad = E_PAD - E
    pad_src = (jnp.arange(pad, dtype=jnp.int32) * 97) % N
    pad_dst = N + jnp.arange(pad, dtype=jnp.int32) % PAD_DST_ROWS
    src_p = jnp.concatenate([src, pad_src]).reshape(NC * NS, CHUNKS, LANES)
    dst_p = jnp.concatenate([dst, pad_dst]).reshape(NC * NS, CHUNKS, LANES)

    P1, Dg = _agg_deg(src_p, dst_p, x)
    Dg = Dg.reshape(NC, N_ACC)
    d0 = Dg[0, :N, None]
    d1 = Dg[1, :N, None]
    h = _linear(P1[0, :N], P1[1, :N], d0, d1, x, W1_l, W1_r, b1, True)
    (P2,) = _agg_nodeg(src_p, dst_p, h)
    return _linear(P2[0, :N], P2[1, :N], d0, d1, h, W2_l, W2_r, b2, False)


# speedup vs baseline: 1.0498x; 1.0498x over previous
import functools

import jax
import jax.numpy as jnp
from jax import lax
from jax.experimental import pallas as pl
from jax.experimental.pallas import tpu as pltpu
from jax.experimental.pallas import tpu_sc as plsc

N = 10000
E = 320000
D = 128

NC = 2
NS = 16
LANES = 128
CHUNKS = 80
E_PAD = NC * NS * CHUNKS * LANES
ROWS_PER_TILE = 632
N_ACC = NC * NS * ROWS_PER_TILE // 2
PAD_DST_ROWS = 64


def _make_agg(with_deg: bool):
    out_type = [jax.ShapeDtypeStruct((NC, N_ACC, D), jnp.float32)]
    if with_deg:
        out_type.append(jax.ShapeDtypeStruct((NC * N_ACC,), jnp.float32))

    scratch = [
        pltpu.VMEM((CHUNKS // 2, LANES), jnp.int32),
        pltpu.VMEM((CHUNKS // 2, LANES), jnp.int32),
        pltpu.VMEM((LANES, D), jnp.float32),
        pltpu.VMEM((LANES, D), jnp.float32),
        pltpu.VMEM((LANES,), jnp.float32),
        pltpu.VMEM((640,), jnp.float32),
        pltpu.VMEM_SHARED((N_ACC, D), jnp.float32),
        pltpu.VMEM_SHARED((N_ACC,), jnp.float32),
        pltpu.SemaphoreType.DMA,
        pltpu.SemaphoreType.DMA,
        pltpu.SemaphoreType.DMA,
        pltpu.SemaphoreType.DMA,
    ]

    def body(src_hbm, dst_hbm, feat_hbm, *rest):
        if with_deg:
            out_hbm, deg_hbm = rest[0], rest[1]
            scratches = rest[2:]
        else:
            out_hbm = rest[0]
            scratches = rest[1:]
        (src_v, dst_v, rows0_v, rows1_v, ones_v, dzero_v, acc_sh, deg_sh,
         gsem0, gsem1, ssem0, ssem1) = scratches
        rows_v = rows0_v

        cid = lax.axis_index("c")
        sid = lax.axis_index("s")
        tid = cid * NS + sid

        def zrow(i, _):
            for j in range(D // 16):
                rows_v[i, pl.ds(j * 16, 16)] = jnp.zeros((16,), jnp.float32)
            return 0
        lax.fori_loop(0, LANES, zrow, 0)
        for j in range(LANES // 16):
            ones_v[pl.ds(j * 16, 16)] = jnp.ones((16,), jnp.float32)

        base = sid * ROWS_PER_TILE
        full, tail = divmod(ROWS_PER_TILE, LANES)
        for k in range(full):
            pltpu.sync_copy(rows_v, acc_sh.at[pl.ds(base + k * LANES, LANES)])
        if tail:
            pltpu.sync_copy(rows_v.at[pl.ds(0, tail)],
                            acc_sh.at[pl.ds(base + full * LANES, tail)])
        if with_deg:
            def zdeg(i, _):
                dzero_v[pl.ds(i * 16, 16)] = jnp.zeros((16,), jnp.float32)
                return 0
            lax.fori_loop(0, 640 // 16, zdeg, 0)
            pltpu.sync_copy(dzero_v.at[pl.ds(0, ROWS_PER_TILE)],
                            deg_sh.at[pl.ds(base, ROWS_PER_TILE)])
        plsc.subcore_barrier()

        bufs = ((rows0_v, gsem0, ssem0), (rows1_v, gsem1, ssem1))
        HALF = CHUNKS // 2
        NPAIR = HALF // 2

        def gather(j, b):
            rv, gs, _ = bufs[b]
            pltpu.async_copy(feat_hbm.at[src_v.at[j]], rv, gs)

        for h in range(2):
            pltpu.sync_copy(src_hbm.at[tid, pl.ds(h * HALF, HALF)], src_v)
            pltpu.sync_copy(dst_hbm.at[tid, pl.ds(h * HALF, HALF)], dst_v)
            gather(0, 0)
            gather(1, 1)

            def pair(i, _):
                j0 = 2 * i
                for b in range(2):
                    j = j0 + b
                    rv, gs, ss = bufs[b]
                    pltpu.make_async_copy(feat_hbm.at[src_v.at[j]], rv,
                                          gs).wait()
                    if with_deg:
                        pltpu.sync_copy(ones_v, deg_sh.at[dst_v.at[j]],
                                        add=True)
                    pltpu.async_copy(rv, acc_sh.at[dst_v.at[j]], ss, add=True)
                for b in range(2):
                    j = j0 + b
                    rv, gs, ss = bufs[b]
                    pltpu.make_async_copy(rv, acc_sh.at[dst_v.at[j]],
                                          ss).wait()

                    @pl.when(i + 1 < NPAIR)
                    def _():
                        gather(j + 2, b)
                return 0

            lax.fori_loop(0, NPAIR, pair, 0)
        plsc.subcore_barrier()

        pltpu.sync_copy(acc_sh.at[pl.ds(base, ROWS_PER_TILE)],
                        out_hbm.at[cid, pl.ds(base, ROWS_PER_TILE)])
        if with_deg:
            pltpu.sync_copy(deg_sh.at[pl.ds(base, ROWS_PER_TILE)],
                            dzero_v.at[pl.ds(0, ROWS_PER_TILE)])
            pltpu.sync_copy(dzero_v.at[pl.ds(0, ROWS_PER_TILE)],
                            deg_hbm.at[pl.ds(cid * N_ACC + base,
                                             ROWS_PER_TILE)])

    mesh = plsc.VectorSubcoreMesh(core_axis_name="c", subcore_axis_name="s")
    return pl.kernel(body, out_type=out_type, mesh=mesh,
                     scratch_types=scratch)


_agg_deg = _make_agg(True)
_agg_nodeg = _make_agg(False)


def _lin_body(relu, p0, p1, d0, d1, xr, wl, wr, b, o):
    deg = jnp.clip(d0[...] + d1[...], 1.0, None)
    mean = (p0[0] + p1[0]) / deg
    y = (jnp.dot(mean, wl[...], preferred_element_type=jnp.float32)
         + jnp.dot(xr[...], wr[...], preferred_element_type=jnp.float32)
         + b[...])
    o[...] = jnp.maximum(y, 0.0) if relu else y


def _linear(P, d0, d1, x, W_l, W_r, b, relu):
    B = 2000
    grid = (N // B,)
    row = lambda i: (i, 0)
    fix = lambda i: (0, 0)
    return pl.pallas_call(
        functools.partial(_lin_body, relu),
        grid=grid,
        in_specs=[
            pl.BlockSpec((1, B, D), lambda i: (0, i, 0)),
            pl.BlockSpec((1, B, D), lambda i: (1, i, 0)),
            pl.BlockSpec((B, 1), row), pl.BlockSpec((B, 1), row),
            pl.BlockSpec((B, D), row),
            pl.BlockSpec((D, D), fix), pl.BlockSpec((D, D), fix),
            pl.BlockSpec((1, D), fix),
        ],
        out_specs=pl.BlockSpec((B, D), row),
        out_shape=jax.ShapeDtypeStruct((N, D), jnp.float32),
    )(P, P, d0, d1, x, W_l, W_r, b.reshape(1, D))


def kernel(x, edge_index, W1_l, W1_r, b1, W2_l, W2_r, b2):
    src = edge_index[0]
    dst = edge_index[1]
    pad = E_PAD - E
    pad_src = (jnp.arange(pad, dtype=jnp.int32) * 97) % N
    pad_dst = N + jnp.arange(pad, dtype=jnp.int32) % PAD_DST_ROWS
    src_p = jnp.concatenate([src, pad_src]).reshape(NC * NS, CHUNKS, LANES)
    dst_p = jnp.concatenate([dst, pad_dst]).reshape(NC * NS, CHUNKS, LANES)

    P1, Dg = _agg_deg(src_p, dst_p, x)
    Dg = Dg.reshape(NC, N_ACC)
    d0 = Dg[0, :N, None]
    d1 = Dg[1, :N, None]
    h = _linear(P1, d0, d1, x, W1_l, W1_r, b1, True)
    (P2,) = _agg_nodeg(src_p, dst_p, h)
    return _linear(P2, d0, d1, h, W2_l, W2_r, b2, False)


# speedup vs baseline: 1.1072x; 1.0547x over previous
import functools

import jax
import jax.numpy as jnp
from jax import lax
from jax.experimental import pallas as pl
from jax.experimental.pallas import tpu as pltpu
from jax.experimental.pallas import tpu_sc as plsc

N = 10000
E = 320000
D = 128
DP = D // 2

NC = 2
NS = 16
LANES = 128
CHUNKS = 80
E_PAD = NC * NS * CHUNKS * LANES
ROWS_PER_TILE = 632
N_ACC = NC * NS * ROWS_PER_TILE // 2
PAD_DST_ROWS = 64


def _make_agg(with_deg: bool):
    out_type = [jax.ShapeDtypeStruct((NC, N_ACC, D), jnp.float32)]
    if with_deg:
        out_type.append(jax.ShapeDtypeStruct((NC * N_ACC,), jnp.float32))

    scratch = [
        pltpu.VMEM((CHUNKS // 2, LANES), jnp.int32),
        pltpu.VMEM((CHUNKS // 2, LANES), jnp.int32),
        pltpu.VMEM((LANES, DP), jnp.int32),
        pltpu.VMEM((LANES, DP), jnp.int32),
        pltpu.VMEM((LANES, D), jnp.float32),
        pltpu.VMEM((LANES,), jnp.float32),
        pltpu.VMEM((640,), jnp.float32),
        pltpu.VMEM_SHARED((N_ACC, D), jnp.float32),
        pltpu.VMEM_SHARED((N_ACC,), jnp.float32),
        pltpu.SemaphoreType.DMA,
        pltpu.SemaphoreType.DMA,
        pltpu.SemaphoreType.DMA,
    ]

    def body(src_hbm, dst_hbm, feat_hbm, *rest):
        if with_deg:
            out_hbm, deg_hbm = rest[0], rest[1]
            scratches = rest[2:]
        else:
            out_hbm = rest[0]
            scratches = rest[1:]
        (src_v, dst_v, pbuf0, pbuf1, fbuf, ones_v, dzero_v,
         acc_sh, deg_sh, gsem0, gsem1, fsem) = scratches
        pbufs = (pbuf0, gsem0), (pbuf1, gsem1)

        cid = lax.axis_index("c")
        sid = lax.axis_index("s")
        tid = cid * NS + sid

        for i in range(LANES):
            for j in range(D // 16):
                fbuf[i, pl.ds(j * 16, 16)] = jnp.zeros((16,), jnp.float32)
        for j in range(LANES // 16):
            ones_v[pl.ds(j * 16, 16)] = jnp.ones((16,), jnp.float32)

        base = sid * ROWS_PER_TILE
        full, tail = divmod(ROWS_PER_TILE, LANES)
        for k in range(full):
            pltpu.sync_copy(fbuf, acc_sh.at[pl.ds(base + k * LANES, LANES)])
        if tail:
            pltpu.sync_copy(fbuf.at[pl.ds(0, tail)],
                            acc_sh.at[pl.ds(base + full * LANES, tail)])
        if with_deg:
            def zdeg(i, _):
                dzero_v[pl.ds(i * 16, 16)] = jnp.zeros((16,), jnp.float32)
                return 0
            lax.fori_loop(0, 640 // 16, zdeg, 0)
            pltpu.sync_copy(dzero_v.at[pl.ds(0, ROWS_PER_TILE)],
                            deg_sh.at[pl.ds(base, ROWS_PER_TILE)])
        plsc.subcore_barrier()

        HALF = CHUNKS // 2
        MASK_HI = jnp.int32(-65536)

        def gather(j, b):
            pv, gs = pbufs[b]
            pltpu.async_copy(feat_hbm.at[src_v.at[j]], pv, gs)

        def scat_start(jj):
            return pltpu.async_copy(fbuf, acc_sh.at[dst_v.at[jj]], fsem,
                                    add=True)

        def scat_wait(jj):
            pltpu.make_async_copy(fbuf, acc_sh.at[dst_v.at[jj]], fsem).wait()

        def unpack(pv):
            @plsc.parallel_loop(0, LANES, 1, unroll=4)
            def row(r):
                for g in range(D // 32):
                    w = pv[r, pl.ds(g * 16, 16)]
                    fv0 = lax.bitcast_convert_type(w << jnp.int32(16),
                                                   jnp.float32)
                    fv1 = lax.bitcast_convert_type(w & MASK_HI, jnp.float32)
                    fbuf[r, pl.ds(g * 32, 16)] = fv0
                    fbuf[r, pl.ds(g * 32 + 16, 16)] = fv1

        for h in range(2):
            pltpu.sync_copy(src_hbm.at[tid, pl.ds(h * HALF, HALF)], src_v)
            pltpu.sync_copy(dst_hbm.at[tid, pl.ds(h * HALF, HALF)], dst_v)
            gather(0, 0)

            def pair(i, _):
                for b in range(2):
                    j = 2 * i + b
                    pv, gs = pbufs[b]
                    pltpu.make_async_copy(feat_hbm.at[src_v.at[j]], pv,
                                          gs).wait()

                    @pl.when(j + 1 < HALF)
                    def _():
                        gather(j + 1, 1 - b)
                    if with_deg:
                        pltpu.sync_copy(ones_v, deg_sh.at[dst_v.at[j]],
                                        add=True)

                    @pl.when(j > 0)
                    def _():
                        scat_wait(j - 1)
                    unpack(pv)
                    scat_start(j)
                return 0

            lax.fori_loop(0, HALF // 2, pair, 0)
            scat_wait(HALF - 1)
        plsc.subcore_barrier()

        pltpu.sync_copy(acc_sh.at[pl.ds(base, ROWS_PER_TILE)],
                        out_hbm.at[cid, pl.ds(base, ROWS_PER_TILE)])
        if with_deg:
            pltpu.sync_copy(deg_sh.at[pl.ds(base, ROWS_PER_TILE)],
                            dzero_v.at[pl.ds(0, ROWS_PER_TILE)])
            pltpu.sync_copy(dzero_v.at[pl.ds(0, ROWS_PER_TILE)],
                            deg_hbm.at[pl.ds(cid * N_ACC + base,
                                             ROWS_PER_TILE)])

    mesh = plsc.VectorSubcoreMesh(core_axis_name="c", subcore_axis_name="s")
    return pl.kernel(body, out_type=out_type, mesh=mesh,
                     scratch_types=scratch,
                     compiler_params=pltpu.CompilerParams(
                         use_tc_tiling_on_sc=False))


_agg_deg = _make_agg(True)
_agg_nodeg = _make_agg(False)


def _pack(feat):
    fb = feat.astype(jnp.bfloat16)
    fr = fb.reshape(N, D // 32, 2, 16).swapaxes(2, 3)
    return lax.bitcast_convert_type(fr, jnp.int32).reshape(N, DP)


def _lin_body(relu, p0, p1, d0, d1, xr, wl, wr, b, o):
    deg = jnp.clip(d0[...] + d1[...], 1.0, None)
    mean = (p0[0] + p1[0]) / deg
    y = (jnp.dot(mean, wl[...], preferred_element_type=jnp.float32)
         + jnp.dot(xr[...], wr[...], preferred_element_type=jnp.float32)
         + b[...])
    o[...] = jnp.maximum(y, 0.0) if relu else y


def _linear(P, d0, d1, x, W_l, W_r, b, relu):
    B = 2000
    grid = (N // B,)
    row = lambda i: (i, 0)
    fix = lambda i: (0, 0)
    return pl.pallas_call(
        functools.partial(_lin_body, relu),
        grid=grid,
        in_specs=[
            pl.BlockSpec((1, B, D), lambda i: (0, i, 0)),
            pl.BlockSpec((1, B, D), lambda i: (1, i, 0)),
            pl.BlockSpec((B, 1), row), pl.BlockSpec((B, 1), row),
            pl.BlockSpec((B, D), row),
            pl.BlockSpec((D, D), fix), pl.BlockSpec((D, D), fix),
            pl.BlockSpec((1, D), fix),
        ],
        out_specs=pl.BlockSpec((B, D), row),
        out_shape=jax.ShapeDtypeStruct((N, D), jnp.float32),
    )(P, P, d0, d1, x, W_l, W_r, b.reshape(1, D))


def kernel(x, edge_index, W1_l, W1_r, b1, W2_l, W2_r, b2):
    src = edge_index[0]
    dst = edge_index[1]
    pad = E_PAD - E
    pad_src = (jnp.arange(pad, dtype=jnp.int32) * 97) % N
    pad_dst = N + jnp.arange(pad, dtype=jnp.int32) % PAD_DST_ROWS
    src_p = jnp.concatenate([src, pad_src]).reshape(NC * NS, CHUNKS, LANES)
    dst_p = jnp.concatenate([dst, pad_dst]).reshape(NC * NS, CHUNKS, LANES)

    P1, Dg = _agg_deg(src_p, dst_p, _pack(x))
    Dg = Dg.reshape(NC, N_ACC)
    d0 = Dg[0, :N, None]
    d1 = Dg[1, :N, None]
    h = _linear(P1, d0, d1, x, W1_l, W1_r, b1, True)
    (P2,) = _agg_nodeg(src_p, dst_p, _pack(h))
    return _linear(P2, d0, d1, h, W2_l, W2_r, b2, False)
